# baseline (speedup 1.0000x reference)
import jax
import jax.numpy as jnp
from jax import lax
from jax.experimental import pallas as pl
from jax.experimental.pallas import tpu as pltpu
from jax.experimental.pallas import tpu_sc as plsc

_B = 4096
_IMG_D = 3 * 64 * 64
_OUT_D = _IMG_D + 16 + 16 + 8

_NC, _NS = 2, 16
_NW = _NC * _NS
_BPW = _B // _NW
_L = 16

_TC_BLOCK = 128


def _sc_gather_body(verb_hbm, noun_hbm, evt_hbm, ent_hbm,
                    ev_out, en_out,
                    vidx, nidx, rv, rn, sem_v, sem_n):
    wid = lax.axis_index("s") * _NC + lax.axis_index("c")
    base = wid * _BPW
    pltpu.sync_copy(verb_hbm.at[pl.ds(base, _BPW)], vidx)
    pltpu.sync_copy(noun_hbm.at[pl.ds(base, _BPW)], nidx)
    for i in range(_BPW // _L):
        sl = pl.ds(i * _L, _L)
        vidx[sl] = lax.shift_right_logical(vidx[sl], 3)
        nidx[sl] = lax.shift_right_logical(nidx[sl], 3)
    cv = pltpu.async_copy(evt_hbm.at[vidx], rv, sem_v)
    cn = pltpu.async_copy(ent_hbm.at[nidx], rn, sem_n)
    cv.wait()
    cn.wait()
    pltpu.sync_copy(rv, ev_out.at[pl.ds(base, _BPW)])
    pltpu.sync_copy(rn, en_out.at[pl.ds(base, _BPW)])


def _sc_gather(verb, noun, evt, ent):
    mesh = plsc.VectorSubcoreMesh(core_axis_name="c", subcore_axis_name="s",
                                  num_cores=_NC, num_subcores=_NS)
    f = pl.kernel(
        _sc_gather_body,
        out_type=[jax.ShapeDtypeStruct((_B, 128), jnp.float32),
                  jax.ShapeDtypeStruct((_B, 128), jnp.float32)],
        mesh=mesh,
        scratch_types=[pltpu.VMEM((_BPW,), jnp.int32),
                       pltpu.VMEM((_BPW,), jnp.int32),
                       pltpu.VMEM((_BPW, 128), jnp.float32),
                       pltpu.VMEM((_BPW, 128), jnp.float32),
                       pltpu.SemaphoreType.DMA,
                       pltpu.SemaphoreType.DMA],
    )
    return f(verb, noun, evt, ent)


def _tc_body(img_ref, ev_ref, en_ref, ec_ref, vlo_ref, nlo_ref, c_ref, out_ref):
    out_ref[:, :_IMG_D] = img_ref[...]
    vlo = vlo_ref[...] & 7
    nlo = nlo_ref[...] & 7
    c = c_ref[...]
    ev = ev_ref[...]
    en = en_ref[...]
    ec = ec_ref[...]
    sel_v = jnp.zeros((img_ref.shape[0], 16), jnp.float32)
    sel_n = jnp.zeros((img_ref.shape[0], 16), jnp.float32)
    for k in range(8):
        sel_v = jnp.where(vlo == k, ev[:, 16 * k:16 * k + 16], sel_v)
        sel_n = jnp.where(nlo == k, en[:, 16 * k:16 * k + 16], sel_n)
    sel_c = jnp.zeros((img_ref.shape[0], 8), jnp.float32)
    for k in range(16):
        sel_c = jnp.where(c == k, ec[:, 8 * k:8 * k + 8], sel_c)
    out_ref[:, _IMG_D:_IMG_D + 16] = sel_v
    out_ref[:, _IMG_D + 16:_IMG_D + 32] = sel_n
    out_ref[:, _IMG_D + 32:_OUT_D] = sel_c


def _tc_concat(img, ev, en, ec_flat, vlo, nlo, color):
    nb = _B // _TC_BLOCK
    return pl.pallas_call(
        _tc_body,
        grid=(nb,),
        in_specs=[pl.BlockSpec((_TC_BLOCK, _IMG_D), lambda i: (i, 0)),
                  pl.BlockSpec((_TC_BLOCK, 128), lambda i: (i, 0)),
                  pl.BlockSpec((_TC_BLOCK, 128), lambda i: (i, 0)),
                  pl.BlockSpec((1, 128), lambda i: (0, 0)),
                  pl.BlockSpec((_TC_BLOCK, 1), lambda i: (i, 0)),
                  pl.BlockSpec((_TC_BLOCK, 1), lambda i: (i, 0)),
                  pl.BlockSpec((_TC_BLOCK, 1), lambda i: (i, 0))],
        out_specs=pl.BlockSpec((_TC_BLOCK, _OUT_D), lambda i: (i, 0)),
        out_shape=jax.ShapeDtypeStruct((_B, _OUT_D), jnp.float32),
    )(img, ev, en, ec_flat, vlo, nlo, color)


def kernel(image, verb, noun, color, emb_v, emb_n, emb_c):
    img = image.astype(jnp.float32).reshape(image.shape[0], -1)
    verb = verb.astype(jnp.int32)
    noun = noun.astype(jnp.int32)
    color = color.astype(jnp.int32)
    evt = emb_v.astype(jnp.float32).reshape(-1, 128)
    ent = emb_n.astype(jnp.float32).reshape(-1, 128)
    ec_flat = emb_c.astype(jnp.float32).reshape(1, 128)
    ev, en = _sc_gather(verb, noun, evt, ent)
    return _tc_concat(img, ev, en, ec_flat,
                      verb.reshape(-1, 1), noun.reshape(-1, 1),
                      color.reshape(-1, 1))
